# Initial kernel scaffold; baseline (speedup 1.0000x reference)
import jax
import jax.numpy as jnp
from jax.experimental import pallas as pl


def kernel(h, batch, W, b):
    raise NotImplementedError("write your pallas kernel here")


# speedup vs baseline: 4.0324x; 4.0324x over previous
import functools

import jax
import jax.numpy as jnp
from jax import lax
from jax.experimental import pallas as pl
from jax.experimental.pallas import tpu as pltpu
from jax.experimental.pallas import tpu_sc as plsc

N = 100000
D = 256
B = 64
NC = 2
NS = 16
NW = NC * NS
RPW = 3128
CH = 136
NCHUNK = RPW // CH
L = 16
KD = D // L
SEG_W = 384
BATCH_BUF = RPW + 24

_NEG_INF = float("-inf")

_mesh = plsc.VectorSubcoreMesh(
    core_axis_name="c", subcore_axis_name="s", num_cores=NC, num_subcores=NS
)


@functools.partial(
    pl.kernel,
    out_type=jax.ShapeDtypeStruct((NW * B, SEG_W), jnp.float32),
    mesh=_mesh,
    compiler_params=pltpu.CompilerParams(needs_layout_passes=False),
    scratch_types=[
        pltpu.VMEM((CH, D), jnp.float32),
        pltpu.VMEM((CH, D), jnp.float32),
        pltpu.VMEM((BATCH_BUF,), jnp.int32),
        pltpu.VMEM((D,), jnp.float32),
        pltpu.VMEM((B, SEG_W), jnp.float32),
        pltpu.SemaphoreType.DMA,
        pltpu.SemaphoreType.DMA,
        pltpu.SemaphoreType.DMA,
    ],
)
def _attn_pool_partials(h_hbm, batch_hbm, w_hbm, out_hbm,
                        hbuf_a, hbuf_b, bbuf, wbuf, accf, sem_a, sem_b, sem_m):
    wid = lax.axis_index("s") * NC + lax.axis_index("c")
    start = wid * RPW
    end = jnp.minimum(start + RPW, N)

    def aligned(x):
        return pl.multiple_of(x, 8)

    def chunk_start(c):
        return aligned(jnp.minimum(start + c * CH, end - CH))

    bload = aligned(jnp.minimum(start, N - RPW))
    cp_b = pltpu.async_copy(
        batch_hbm.at[pl.ds(bload, RPW)], bbuf.at[pl.ds(0, RPW)], sem_m
    )
    hbufs = (hbuf_a, hbuf_b)
    sems = (sem_a, sem_b)
    cps = [
        pltpu.async_copy(h_hbm.at[pl.ds(chunk_start(0), CH)], hbuf_a, sem_a),
        pltpu.async_copy(h_hbm.at[pl.ds(chunk_start(1), CH)], hbuf_b, sem_b),
    ]

    zv = jnp.zeros((L,), jnp.float32)
    ninf = jnp.full((L,), _NEG_INF, jnp.float32)

    def init_body(s, _):
        for k in range(KD):
            accf[s, pl.ds(k * L, L)] = zv
        accf[s, pl.ds(D, L)] = ninf
        accf[s, pl.ds(D + L, L)] = zv
        return 0

    lax.fori_loop(0, B, init_body, 0)

    cp_b.wait()
    cp_w = pltpu.async_copy(w_hbm, wbuf, sem_m)
    cp_w.wait()
    wvecs = [wbuf[pl.ds(k * L, L)] for k in range(KD)]
    lane = lax.iota(jnp.int32, L)


    def flush(cur, m, dv, accs):
        for k in range(KD):
            accf[cur, pl.ds(k * L, L)] = accs[k]
        accf[cur, pl.ds(D, L)] = jnp.full((L,), m, jnp.float32)
        accf[cur, pl.ds(D + L, L)] = dv

    def make_row_body(hbuf, chunk_boff):
        def row_body(j, carry):
            cur, m, dv, accs = carry[0], carry[1], carry[2], carry[3:]
            seg = bbuf[pl.ds(chunk_boff + j, L)][0]
            hv = [hbuf[j, pl.ds(k * L, L)] for k in range(KD)]
            parts = [hv[k] * wvecs[k] for k in range(KD)]
            while len(parts) > 1:
                parts = [parts[i] + parts[i + 1] for i in range(0, len(parts), 2)]
            s = jnp.sum(parts[0])

            def on_switch(cur_, m_, dv_, accs_):
                flush(cur_, m_, dv_, accs_)
                return (jnp.float32(_NEG_INF), zv) + tuple(zv for _ in range(KD))

            def keep(cur_, m_, dv_, accs_):
                return (m_, dv_) + tuple(accs_)

            res = lax.cond(seg != cur, on_switch, keep, cur, m, dv, accs)
            m, dv, accs = res[0], res[1], res[2:]

            m2 = jnp.maximum(m, s)
            corr = jnp.exp(jnp.full((L,), m - m2, jnp.float32))
            p = jnp.exp(jnp.full((L,), s - m2, jnp.float32))
            dv = dv * corr + p
            accs = tuple(a * corr + p * h for a, h in zip(accs, hv))
            return (seg, m2, dv) + accs

        return row_body

    boff0 = start - bload
    carry = (bbuf[pl.ds(boff0, L)][0], jnp.float32(_NEG_INF), zv) + tuple(
        zv for _ in range(KD)
    )
    for c in range(NCHUNK):
        cs = chunk_start(c)
        jstart = (start + c * CH) - cs
        cps[c % 2].wait()
        carry = lax.fori_loop(
            jstart, CH, make_row_body(hbufs[c % 2], cs - bload), carry
        )
        if c + 2 < NCHUNK:
            cps[c % 2] = pltpu.async_copy(
                h_hbm.at[pl.ds(chunk_start(c + 2), CH)], hbufs[c % 2], sems[c % 2]
            )

    flush(carry[0], carry[1], carry[2], carry[3:])
    pltpu.sync_copy(accf, out_hbm.at[pl.ds(aligned(wid * B), B)])


@functools.partial(
    pl.kernel,
    out_type=jax.ShapeDtypeStruct((B * D,), jnp.float32),
    mesh=_mesh,
    compiler_params=pltpu.CompilerParams(needs_layout_passes=False),
    scratch_types=[
        pltpu.VMEM((NW,), jnp.int32),
        pltpu.VMEM((NW, SEG_W), jnp.float32),
        pltpu.VMEM((D,), jnp.float32),
        pltpu.SemaphoreType.DMA,
    ],
)
def _attn_pool_combine(part_hbm, out_hbm, idx_v, rows_v, obuf, sem):
    wid = lax.axis_index("s") * NC + lax.axis_index("c")
    iota = lax.iota(jnp.int32, L)
    for rep in range(B // NW):
        s = wid + NW * rep
        idx_v[pl.ds(0, L)] = iota * B + s
        idx_v[pl.ds(L, L)] = (iota + L) * B + s
        pltpu.async_copy(part_hbm.at[idx_v], rows_v, sem).wait()

        mv = rows_v[0, pl.ds(D, L)]
        for w in range(1, NW):
            mv = jnp.maximum(mv, rows_v[w, pl.ds(D, L)])
        mg = jnp.where(mv == _NEG_INF, jnp.zeros_like(mv), mv)

        def comb_body(w, carry):
            den, nums = carry[0], carry[1:]
            cv = jnp.exp(rows_v[w, pl.ds(D, L)] - mg)
            den = den + cv * rows_v[w, pl.ds(D + L, L)]
            nums = tuple(
                n + cv * rows_v[w, pl.ds(k * L, L)] for k, n in enumerate(nums)
            )
            return (den,) + nums

        zv = jnp.zeros((L,), jnp.float32)
        res = lax.fori_loop(0, NW, comb_body, (zv,) * (KD + 1))
        den, nums = res[0], res[1:]
        den = jnp.where(den > 0.0, den, jnp.ones_like(den))
        for k in range(KD):
            obuf[pl.ds(k * L, L)] = nums[k] / den
        pltpu.sync_copy(obuf, out_hbm.at[pl.ds(pl.multiple_of(s * D, 8), D)])


def kernel(h, batch, W, b):
    del b
    batch32 = batch.astype(jnp.int32)
    wv = W.reshape(D)
    partials = _attn_pool_partials(h, batch32, wv)
    return _attn_pool_combine(partials).reshape(B, D)


# speedup vs baseline: 4.1823x; 1.0372x over previous
import functools

import jax
import jax.numpy as jnp
from jax import lax
from jax.experimental import pallas as pl
from jax.experimental.pallas import tpu as pltpu
from jax.experimental.pallas import tpu_sc as plsc

N = 100000
D = 256
B = 64
NC = 2
NS = 16
NW = NC * NS
RPW = 3128
CH = 136
NCHUNK = RPW // CH
L = 16
KD = D // L
SEG_W = 384
BATCH_BUF = RPW + 24
BBOUND = B + 24

_NEG_INF = float("-inf")

_mesh = plsc.VectorSubcoreMesh(
    core_axis_name="c", subcore_axis_name="s", num_cores=NC, num_subcores=NS
)


@functools.partial(
    pl.kernel,
    out_type=jax.ShapeDtypeStruct((NW * B, SEG_W), jnp.float32),
    mesh=_mesh,
    compiler_params=pltpu.CompilerParams(needs_layout_passes=False),
    scratch_types=[
        pltpu.VMEM((CH, D), jnp.float32),
        pltpu.VMEM((CH, D), jnp.float32),
        pltpu.VMEM((BATCH_BUF,), jnp.int32),
        pltpu.VMEM((BBOUND,), jnp.int32),
        pltpu.VMEM((D,), jnp.float32),
        pltpu.VMEM((B, SEG_W), jnp.float32),
        pltpu.SemaphoreType.DMA,
        pltpu.SemaphoreType.DMA,
        pltpu.SemaphoreType.DMA,
    ],
)
def _attn_pool_partials(h_hbm, batch_hbm, starts_hbm, w_hbm, out_hbm,
                        hbuf_a, hbuf_b, bbuf, sbuf, wbuf, accf,
                        sem_a, sem_b, sem_m):
    wid = lax.axis_index("s") * NC + lax.axis_index("c")
    start = wid * RPW
    end = jnp.minimum(start + RPW, N)

    def aligned(x):
        return pl.multiple_of(x, 8)

    def chunk_start(c):
        return aligned(jnp.minimum(start + c * CH, end - CH))

    bload = aligned(jnp.minimum(start, N - RPW))
    cp_b = pltpu.async_copy(
        batch_hbm.at[pl.ds(bload, RPW)], bbuf.at[pl.ds(0, RPW)], sem_m
    )
    hbufs = (hbuf_a, hbuf_b)
    sems = (sem_a, sem_b)
    cps = [
        pltpu.async_copy(h_hbm.at[pl.ds(chunk_start(0), CH)], hbuf_a, sem_a),
        pltpu.async_copy(h_hbm.at[pl.ds(chunk_start(1), CH)], hbuf_b, sem_b),
    ]

    zv = jnp.zeros((L,), jnp.float32)
    ninf = jnp.full((L,), _NEG_INF, jnp.float32)

    def init_body(s, _):
        for k in range(KD):
            accf[s, pl.ds(k * L, L)] = zv
        accf[s, pl.ds(D, L)] = ninf
        accf[s, pl.ds(D + L, L)] = zv
        return 0

    lax.fori_loop(0, B, init_body, 0)

    cp_b.wait()
    cp_s = pltpu.async_copy(starts_hbm.at[pl.ds(0, B + 8)], sbuf.at[pl.ds(0, B + 8)], sem_m)
    cp_s.wait()
    cp_w = pltpu.async_copy(w_hbm, wbuf, sem_m)
    cp_w.wait()
    wvecs = [wbuf[pl.ds(k * L, L)] for k in range(KD)]

    def rd(ref, i):
        return ref[pl.ds(i, L)][0]


    def flush(cur, m, dv, accs):
        for k in range(KD):
            accf[cur, pl.ds(k * L, L)] = accs[k]
        accf[cur, pl.ds(D, L)] = jnp.full((L,), m, jnp.float32)
        accf[cur, pl.ds(D + L, L)] = dv

    def make_row_body(hbuf):
        def row_body(j, carry):
            m, dv, accs = carry[0], carry[1], carry[2:]
            hv = [hbuf[j, pl.ds(k * L, L)] for k in range(KD)]
            parts = [hv[k] * wvecs[k] for k in range(KD)]
            while len(parts) > 1:
                parts = [parts[i] + parts[i + 1] for i in range(0, len(parts), 2)]
            s = jnp.sum(parts[0])
            m2 = jnp.maximum(m, s)
            corr = jnp.exp(jnp.full((L,), m - m2, jnp.float32))
            p = jnp.exp(jnp.full((L,), s - m2, jnp.float32))
            dv = dv * corr + p
            accs = tuple(a * corr + p * h for a, h in zip(accs, hv))
            return (m2, dv) + accs

        return row_body

    def make_run_body(hbuf, wb, cs):
        row_body = make_row_body(hbuf)

        def run_body(carry):
            r, cur = carry[0], carry[1]
            m, dv, accs = carry[2], carry[3], carry[4:]
            e = jnp.clip(rd(sbuf, cur + 1) - cs, r, CH)
            res = lax.fori_loop(r, e, row_body, (m, dv) + accs)
            m, dv, accs = res[0], res[1], res[2:]

            def on_end(cur_, m_, dv_, accs_):
                flush(cur_, m_, dv_, accs_)
                nc = rd(bbuf, wb + e)
                return (nc, jnp.float32(_NEG_INF), zv) + tuple(
                    zv for _ in range(KD)
                )

            def keep(cur_, m_, dv_, accs_):
                return (cur_, m_, dv_) + tuple(accs_)

            res2 = lax.cond(e < CH, on_end, keep, cur, m, dv, accs)
            return (e,) + res2

        return run_body

    def process_chunk(carry, hbuf, cs, jstart):
        run_body = make_run_body(hbuf, cs - bload, cs)
        return lax.while_loop(
            lambda carry: carry[0] < CH, run_body, (jstart,) + carry
        )[1:]

    def wait_chunk(buf_i, c):
        pltpu.make_async_copy(
            h_hbm.at[pl.ds(chunk_start(c), CH)], hbufs[buf_i], sems[buf_i]
        ).wait()

    def fetch_chunk(buf_i, c):
        pltpu.async_copy(
            h_hbm.at[pl.ds(chunk_start(c), CH)], hbufs[buf_i], sems[buf_i]
        )

    def chunk_pair(i, carry):
        c0 = 2 * i
        wait_chunk(0, c0)
        carry = process_chunk(carry, hbuf_a, chunk_start(c0), 0)
        fetch_chunk(0, jnp.minimum(c0 + 2, NCHUNK - 1))
        wait_chunk(1, c0 + 1)
        carry = process_chunk(carry, hbuf_b, chunk_start(c0 + 1), 0)
        fetch_chunk(1, jnp.minimum(c0 + 3, NCHUNK - 1))
        return carry

    boff0 = start - bload
    carry = (bbuf[pl.ds(boff0, L)][0], jnp.float32(_NEG_INF), zv) + tuple(
        zv for _ in range(KD)
    )
    carry = lax.fori_loop(0, (NCHUNK - 1) // 2, chunk_pair, carry)

    cs_last = chunk_start(NCHUNK - 1)
    wait_chunk(0, NCHUNK - 1)
    carry = process_chunk(
        carry, hbuf_a, cs_last, (start + (NCHUNK - 1) * CH) - cs_last
    )
    wait_chunk(1, NCHUNK - 1)

    flush(carry[0], carry[1], carry[2], carry[3:])
    pltpu.sync_copy(accf, out_hbm.at[pl.ds(aligned(wid * B), B)])


@functools.partial(
    pl.kernel,
    out_type=jax.ShapeDtypeStruct((B * D,), jnp.float32),
    mesh=_mesh,
    compiler_params=pltpu.CompilerParams(needs_layout_passes=False),
    scratch_types=[
        pltpu.VMEM((NW,), jnp.int32),
        pltpu.VMEM((NW, SEG_W), jnp.float32),
        pltpu.VMEM((D,), jnp.float32),
        pltpu.SemaphoreType.DMA,
    ],
)
def _attn_pool_combine(part_hbm, out_hbm, idx_v, rows_v, obuf, sem):
    wid = lax.axis_index("s") * NC + lax.axis_index("c")
    iota = lax.iota(jnp.int32, L)
    for rep in range(B // NW):
        s = wid + NW * rep
        idx_v[pl.ds(0, L)] = iota * B + s
        idx_v[pl.ds(L, L)] = (iota + L) * B + s
        pltpu.async_copy(part_hbm.at[idx_v], rows_v, sem).wait()

        mv = rows_v[0, pl.ds(D, L)]
        for w in range(1, NW):
            mv = jnp.maximum(mv, rows_v[w, pl.ds(D, L)])
        mg = jnp.where(mv == _NEG_INF, jnp.zeros_like(mv), mv)

        def comb_body(w, carry):
            den, nums = carry[0], carry[1:]
            cv = jnp.exp(rows_v[w, pl.ds(D, L)] - mg)
            den = den + cv * rows_v[w, pl.ds(D + L, L)]
            nums = tuple(
                n + cv * rows_v[w, pl.ds(k * L, L)] for k, n in enumerate(nums)
            )
            return (den,) + nums

        zv = jnp.zeros((L,), jnp.float32)
        res = lax.fori_loop(0, NW, comb_body, (zv,) * (KD + 1))
        den, nums = res[0], res[1:]
        den = jnp.where(den > 0.0, den, jnp.ones_like(den))
        for k in range(KD):
            obuf[pl.ds(k * L, L)] = nums[k] / den
        pltpu.sync_copy(obuf, out_hbm.at[pl.ds(pl.multiple_of(s * D, 8), D)])


def kernel(h, batch, W, b):
    del b
    batch32 = batch.astype(jnp.int32)
    wv = W.reshape(D)
    starts = jnp.searchsorted(
        batch32, jnp.arange(B + 8, dtype=jnp.int32)
    ).astype(jnp.int32)
    partials = _attn_pool_partials(h, batch32, starts, wv)
    return _attn_pool_combine(partials).reshape(B, D)
